# baseline (speedup 1.0000x reference)
import functools

import jax
import jax.numpy as jnp
from jax import lax
from jax.experimental import pallas as pl
from jax.experimental.pallas import tpu as pltpu
from jax.experimental.pallas import tpu_sc as plsc

NBLK = 512
B = 8
C = 512
D = 3
N = 4096
R = C * D
KK = B // 2
NEG = float(-3.0e38)
IMIN = -2147483647


def _score_body(fx_ref, fy_ref, s_ref):
    X2 = fx_ref[0]
    Y2 = fy_ref[0]
    X = X2.reshape(R // 24, 3, 8, NBLK)
    p_i = lax.broadcasted_iota(jnp.int32, (1, 3, 8, 1), 1)
    s_i = lax.broadcasted_iota(jnp.int32, (1, 3, 8, 1), 2)
    res = (2 * p_i + s_i) % 3

    def resid_sums(a4, wanted):
        return {dd: jnp.sum(jnp.where(res == dd, a4, 0.0), axis=(0, 1, 2))
                for dd in wanted}

    sax = jnp.sum(X, axis=0, keepdims=True)
    Y = Y2.reshape(R // 24, 3, 8, NBLK)
    say = jnp.sum(Y, axis=0, keepdims=True)
    sx = resid_sums(sax, range(3))
    sy = resid_sums(say, range(3))

    m = {}
    for o in (-2, -1, 0, 1, 2):
        if o == 0:
            yo = Y2
        elif o > 0:
            yo = jnp.concatenate(
                [Y2[o:], jnp.zeros((o, NBLK), jnp.float32)], axis=0)
        else:
            yo = jnp.concatenate(
                [jnp.zeros((-o, NBLK), jnp.float32), Y2[:R + o]], axis=0)
        pa = jnp.sum(X * yo.reshape(R // 24, 3, 8, NBLK), axis=0,
                     keepdims=True)
        wanted = [dd for dd in range(3) if 0 <= dd + o < 3]
        pr = resid_sums(pa, wanted)
        for dd in wanted:
            m[(dd, dd + o)] = pr[dd]

    inv_c = jnp.float32(1.0 / C)
    mx = [sx[dd] * inv_c for dd in range(3)]
    my = [sy[dd] * inv_c for dd in range(3)]
    nx = jnp.sqrt(mx[0] * mx[0] + mx[1] * mx[1] + mx[2] * mx[2]) \
        + jnp.float32(1e-6)
    ny = jnp.sqrt(my[0] * my[0] + my[1] * my[1] + my[2] * my[2]) \
        + jnp.float32(1e-6)
    ux = [t / nx for t in mx]
    uy = [t / ny for t in my]
    s = None
    for (dd, dp), mv in m.items():
        term = ux[dd] * uy[dp] * mv
        s = term if s is None else s + term
    s_ref[0, 0, :] = s


def _scores(fxr, fyr):
    return pl.pallas_call(
        _score_body,
        grid=(B, N // NBLK),
        in_specs=[
            pl.BlockSpec((1, R, NBLK), lambda b, nb: (b, 0, nb)),
            pl.BlockSpec((1, R, NBLK), lambda b, nb: (b, 0, nb)),
        ],
        out_specs=pl.BlockSpec((1, 1, NBLK), lambda b, nb: (b, 0, nb)),
        out_shape=jax.ShapeDtypeStruct((B, 1, N), jnp.float32),
        compiler_params=pltpu.CompilerParams(
            dimension_semantics=("arbitrary", "arbitrary")),
    )(fxr, fyr)


def _sc_topk_gather(s, fxr, fyr):
    mesh = plsc.VectorSubcoreMesh(core_axis_name="c", subcore_axis_name="s")
    chunk_len = N // 4

    @functools.partial(
        pl.kernel,
        mesh=mesh,
        out_type=jax.ShapeDtypeStruct((B, 2, KK, R), jnp.float32),
        scratch_types=[
            pltpu.VMEM((chunk_len,), jnp.float32),
            pltpu.VMEM((16,), jnp.float32),
            pltpu.VMEM((16,), jnp.int32),
            pltpu.VMEM((4, 16), jnp.float32),
            pltpu.VMEM((4, 16), jnp.int32),
            pltpu.VMEM((R, 16), jnp.float32),
            pltpu.VMEM((R, 16), jnp.float32),
            pltpu.VMEM((R,), jnp.float32),
            pltpu.VMEM((R,), jnp.float32),
            pltpu.VMEM_SHARED((4, 4, 16), jnp.float32),
            pltpu.VMEM_SHARED((4, 4, 16), jnp.int32),
            pltpu.VMEM_SHARED((4, 16), jnp.int32),
            pltpu.SemaphoreType.DMA,
        ],
        compiler_params=pltpu.CompilerParams(
            use_tc_tiling_on_sc=False, needs_layout_passes=False),
    )
    def k(s_hbm, fx_hbm, fy_hbm, o_hbm,
          sv, cand_v, cand_i, merge_v, merge_i, slab_x, slab_y,
          col_x, col_y, shv, shi, shf, sem):
        ci = lax.axis_index("c")
        si = lax.axis_index("s")
        bl = si // 4
        chunk = si % 4
        batch = ci * 4 + bl
        base = chunk * chunk_len
        pltpu.sync_copy(s_hbm.at[batch, pl.ds(base, chunk_len)], sv)
        lane = lax.iota(jnp.int32, 16)

        def top4(load_row, nrows, fori):
            winners_v, winners_i = [], []
            for _ in range(4):
                prev = list(winners_i)

                def body(j, carry, prev=prev):
                    mv, mi = carry
                    v, iv = load_row(j)
                    for w in prev:
                        v = jnp.where(iv == w, NEG, v)
                    gt = v > mv
                    return jnp.where(gt, v, mv), jnp.where(gt, iv, mi)

                mv = jnp.full((16,), NEG, jnp.float32)
                mi = jnp.zeros((16,), jnp.int32)
                if fori:
                    mv, mi = lax.fori_loop(0, nrows, body, (mv, mi))
                else:
                    for j in range(nrows):
                        mv, mi = body(j, (mv, mi))
                best = jnp.max(mv)
                bi = -jnp.max(jnp.where(mv == best, -mi, jnp.int32(IMIN)))
                winners_v.append(best)
                winners_i.append(bi)
            return winners_v, winners_i

        def load_chunk(j):
            v = sv[pl.ds(j * 16, 16)]
            iv = base + j * 16 + lane
            return v, iv

        wv, wi = top4(load_chunk, chunk_len // 16, fori=True)
        cv = jnp.full((16,), NEG, jnp.float32)
        civ = jnp.zeros((16,), jnp.int32)
        for k4 in range(4):
            cv = jnp.where(lane == k4, wv[k4], cv)
            civ = jnp.where(lane == k4, wi[k4], civ)
        cand_v[...] = cv
        cand_i[...] = civ
        pltpu.sync_copy(cand_v, shv.at[bl, chunk])
        pltpu.sync_copy(cand_i, shi.at[bl, chunk])
        plsc.subcore_barrier()

        @pl.when(chunk == 0)
        def _():
            pltpu.sync_copy(shv.at[bl], merge_v)
            pltpu.sync_copy(shi.at[bl], merge_i)

            def load_merge(j):
                return merge_v[j], merge_i[j]

            _, fin = top4(load_merge, 4, fori=False)
            fiv = jnp.zeros((16,), jnp.int32)
            for k4 in range(KK):
                fiv = jnp.where(lane == k4, fin[k4], fiv)
            cand_i[...] = fiv
            pltpu.sync_copy(cand_i, shf.at[bl])

        plsc.subcore_barrier()
        pltpu.sync_copy(shf.at[bl], cand_i)
        idx16 = cand_i[...]
        my_idx = jnp.max(jnp.where(lane == chunk, idx16, jnp.int32(IMIN)))
        abase = (my_idx // 16) * 16
        coln = my_idx - abase
        cpx = pltpu.async_copy(
            fx_hbm.at[batch, :, pl.ds(abase, 16)], slab_x, sem)
        cpy = pltpu.async_copy(
            fy_hbm.at[batch, :, pl.ds(abase, 16)], slab_y, sem)
        cpx.wait()
        cpy.wait()
        cols = jnp.broadcast_to(coln, (16,))

        @pl.loop(0, R // 16)
        def _(j):
            rows = j * 16 + lane
            col_x[pl.ds(j * 16, 16)] = plsc.load_gather(slab_x, [rows, cols])
            col_y[pl.ds(j * 16, 16)] = plsc.load_gather(slab_y, [rows, cols])

        pltpu.sync_copy(col_x, o_hbm.at[batch, 0, chunk])
        pltpu.sync_copy(col_y, o_hbm.at[batch, 1, chunk])

    return k(s, fxr, fyr)


def kernel(fx, fy, topk):
    b, c, d, n = fx.shape
    assert (b, c, d, n) == (B, C, D, N), (b, c, d, n)
    fxr = fx.reshape(b, c * d, n)
    fyr = fy.reshape(b, c * d, n)
    s = _scores(fxr, fyr).reshape(b, n)
    g = _sc_topk_gather(s, fxr, fyr)
    gx = g[:, 0].transpose(0, 2, 1)
    gy = g[:, 1].transpose(0, 2, 1)
    return gx.reshape(b, c, d, KK), gy.reshape(b, c, d, KK)


# speedup vs baseline: 3.4396x; 3.4396x over previous
import functools

import jax
import jax.numpy as jnp
from jax import lax
from jax.experimental import pallas as pl
from jax.experimental.pallas import tpu as pltpu
from jax.experimental.pallas import tpu_sc as plsc

NBLK = 128
B = 8
C = 512
D = 3
N = 4096
KK = B // 2
NEG = float(-3.0e38)
IMIN = -2147483647


def _score_body(fx_ref, fy_ref, s_ref):
    fxb = fx_ref[0]
    fyb = fy_ref[0]
    acc_m = jnp.zeros((3, 3, 8, NBLK), jnp.float32)
    acc_x = jnp.zeros((3, 8, NBLK), jnp.float32)
    acc_y = jnp.zeros((3, 8, NBLK), jnp.float32)
    for a in range(C // 8):
        xa = fxb[:, 8 * a:8 * a + 8, :]
        ya = fyb[:, 8 * a:8 * a + 8, :]
        acc_m = acc_m + xa[:, None] * ya[None, :]
        acc_x = acc_x + xa
        acc_y = acc_y + ya
    m = jnp.sum(acc_m, axis=2)
    sx = jnp.sum(acc_x, axis=1)
    sy = jnp.sum(acc_y, axis=1)
    inv_c = jnp.float32(1.0 / C)
    mx = sx * inv_c
    my = sy * inv_c
    nx = jnp.sqrt(jnp.sum(mx * mx, axis=0)) + jnp.float32(1e-6)
    ny = jnp.sqrt(jnp.sum(my * my, axis=0)) + jnp.float32(1e-6)
    ux = mx / nx
    uy = my / ny
    s = jnp.sum(ux[:, None, :] * uy[None, :, :] * m, axis=(0, 1))
    s_ref[0, 0, :] = s


def _scores(fxt, fyt):
    return pl.pallas_call(
        _score_body,
        grid=(B, N // NBLK),
        in_specs=[
            pl.BlockSpec((1, D, C, NBLK), lambda b, nb: (b, 0, 0, nb)),
            pl.BlockSpec((1, D, C, NBLK), lambda b, nb: (b, 0, 0, nb)),
        ],
        out_specs=pl.BlockSpec((1, 1, NBLK), lambda b, nb: (b, 0, nb)),
        out_shape=jax.ShapeDtypeStruct((B, 1, N), jnp.float32),
        compiler_params=pltpu.CompilerParams(
            dimension_semantics=("arbitrary", "arbitrary")),
    )(fxt, fyt)


def _sc_topk(s):
    mesh = plsc.VectorSubcoreMesh(core_axis_name="c", subcore_axis_name="s")
    chunk_len = N // 4

    @functools.partial(
        pl.kernel,
        mesh=mesh,
        out_type=jax.ShapeDtypeStruct((B, 16), jnp.int32),
        scratch_types=[
            pltpu.VMEM((chunk_len,), jnp.float32),
            pltpu.VMEM((16,), jnp.float32),
            pltpu.VMEM((16,), jnp.int32),
            pltpu.VMEM((4, 16), jnp.float32),
            pltpu.VMEM((4, 16), jnp.int32),
            pltpu.VMEM_SHARED((4, 4, 16), jnp.float32),
            pltpu.VMEM_SHARED((4, 4, 16), jnp.int32),
        ],
        compiler_params=pltpu.CompilerParams(
            use_tc_tiling_on_sc=False, needs_layout_passes=False),
    )
    def k(s_hbm, oi_hbm,
          sv, cand_v, cand_i, merge_v, merge_i, shv, shi):
        ci = lax.axis_index("c")
        si = lax.axis_index("s")
        bl = si // 4
        chunk = si % 4
        batch = ci * 4 + bl
        base = chunk * chunk_len
        pltpu.sync_copy(s_hbm.at[batch, pl.ds(base, chunk_len)], sv)
        lane = lax.iota(jnp.int32, 16)

        def top4(load_row, nrows, fori):
            winners_v, winners_i = [], []
            for _ in range(4):
                prev = list(winners_i)

                def body(j, carry, prev=prev):
                    mv, mi = carry
                    v, iv = load_row(j)
                    for w in prev:
                        v = jnp.where(iv == w, NEG, v)
                    gt = v > mv
                    return jnp.where(gt, v, mv), jnp.where(gt, iv, mi)

                mv = jnp.full((16,), NEG, jnp.float32)
                mi = jnp.zeros((16,), jnp.int32)
                if fori:
                    mv, mi = lax.fori_loop(0, nrows, body, (mv, mi))
                else:
                    for j in range(nrows):
                        mv, mi = body(j, (mv, mi))
                best = jnp.max(mv)
                bi = -jnp.max(jnp.where(mv == best, -mi, jnp.int32(IMIN)))
                winners_v.append(best)
                winners_i.append(bi)
            return winners_v, winners_i

        def load_chunk(j):
            v = sv[pl.ds(j * 16, 16)]
            iv = base + j * 16 + lane
            return v, iv

        wv, wi = top4(load_chunk, chunk_len // 16, fori=True)
        cv = jnp.full((16,), NEG, jnp.float32)
        civ = jnp.zeros((16,), jnp.int32)
        for k4 in range(4):
            cv = jnp.where(lane == k4, wv[k4], cv)
            civ = jnp.where(lane == k4, wi[k4], civ)
        cand_v[...] = cv
        cand_i[...] = civ
        pltpu.sync_copy(cand_v, shv.at[bl, chunk])
        pltpu.sync_copy(cand_i, shi.at[bl, chunk])
        plsc.subcore_barrier()

        @pl.when(chunk == 0)
        def _():
            pltpu.sync_copy(shv.at[bl], merge_v)
            pltpu.sync_copy(shi.at[bl], merge_i)

            def load_merge(j):
                return merge_v[j], merge_i[j]

            _, fin = top4(load_merge, 4, fori=False)
            fiv = jnp.zeros((16,), jnp.int32)
            for k4 in range(KK):
                fiv = jnp.where(lane == k4, fin[k4], fiv)
            cand_i[...] = fiv
            pltpu.sync_copy(cand_i, oi_hbm.at[batch])

    return k(s)


def _gather_body(idx_ref, fx_ref, fy_ref, ox_ref, oy_ref):
    bb = pl.program_id(0)
    kk = pl.program_id(1)
    col = lax.rem(idx_ref[bb, kk], 128)
    lanei = lax.broadcasted_iota(jnp.int32, (1, 1, 128), 2)
    sel = lanei == col
    ox_ref[0, 0] = jnp.sum(jnp.where(sel, fx_ref[0], 0.0), axis=2)
    oy_ref[0, 0] = jnp.sum(jnp.where(sel, fy_ref[0], 0.0), axis=2)


def _gather(idx, fxt, fyt):
    grid_spec = pltpu.PrefetchScalarGridSpec(
        num_scalar_prefetch=1,
        grid=(B, KK),
        in_specs=[
            pl.BlockSpec((1, D, C, 128),
                         lambda b, k, idx: (b, 0, 0, idx[b, k] // 128)),
            pl.BlockSpec((1, D, C, 128),
                         lambda b, k, idx: (b, 0, 0, idx[b, k] // 128)),
        ],
        out_specs=[
            pl.BlockSpec((1, 1, D, C), lambda b, k, idx: (b, k, 0, 0)),
            pl.BlockSpec((1, 1, D, C), lambda b, k, idx: (b, k, 0, 0)),
        ],
    )
    return pl.pallas_call(
        _gather_body,
        grid_spec=grid_spec,
        out_shape=[jax.ShapeDtypeStruct((B, KK, D, C), jnp.float32),
                   jax.ShapeDtypeStruct((B, KK, D, C), jnp.float32)],
    )(idx, fxt, fyt)


def kernel(fx, fy, topk):
    b, c, d, n = fx.shape
    assert (b, c, d, n) == (B, C, D, N), (b, c, d, n)
    fxt = fx.transpose(0, 2, 1, 3)
    fyt = fy.transpose(0, 2, 1, 3)
    s = _scores(fxt, fyt).reshape(b, n)
    idx = _sc_topk(s)[:, :KK]
    gx, gy = _gather(idx, fxt, fyt)
    return gx.transpose(0, 3, 2, 1), gy.transpose(0, 3, 2, 1)


# speedup vs baseline: 5.3080x; 1.5432x over previous
import functools

import jax
import jax.numpy as jnp
from jax import lax
from jax.experimental import pallas as pl
from jax.experimental.pallas import tpu as pltpu
from jax.experimental.pallas import tpu_sc as plsc

NBLK = 512
B = 8
C = 512
D = 3
N = 4096
KK = B // 2
NEG = float(-3.0e38)
IMIN = -2147483647


def _score_body(fx_ref, fy_ref, s_ref):
    for sub in range(NBLK // 128):
        lo = sub * 128
        fxb = fx_ref[0, :, :, lo:lo + 128]
        fyb = fy_ref[0, :, :, lo:lo + 128]
        acc_m = jnp.zeros((3, 3, 8, 128), jnp.float32)
        acc_x = jnp.zeros((3, 8, 128), jnp.float32)
        acc_y = jnp.zeros((3, 8, 128), jnp.float32)
        for a in range(C // 8):
            xa = fxb[:, 8 * a:8 * a + 8, :]
            ya = fyb[:, 8 * a:8 * a + 8, :]
            acc_m = acc_m + xa[:, None] * ya[None, :]
            acc_x = acc_x + xa
            acc_y = acc_y + ya
        m = jnp.sum(acc_m, axis=2)
        sx = jnp.sum(acc_x, axis=1)
        sy = jnp.sum(acc_y, axis=1)
        inv_c = jnp.float32(1.0 / C)
        mx = sx * inv_c
        my = sy * inv_c
        nx = jnp.sqrt(jnp.sum(mx * mx, axis=0)) + jnp.float32(1e-6)
        ny = jnp.sqrt(jnp.sum(my * my, axis=0)) + jnp.float32(1e-6)
        ux = mx / nx
        uy = my / ny
        s = jnp.sum(ux[:, None, :] * uy[None, :, :] * m, axis=(0, 1))
        s_ref[0, 0, lo:lo + 128] = s


def _scores(fxt, fyt):
    return pl.pallas_call(
        _score_body,
        grid=(B, N // NBLK),
        in_specs=[
            pl.BlockSpec((1, D, C, NBLK), lambda b, nb: (b, 0, 0, nb)),
            pl.BlockSpec((1, D, C, NBLK), lambda b, nb: (b, 0, 0, nb)),
        ],
        out_specs=pl.BlockSpec((1, 1, NBLK), lambda b, nb: (b, 0, nb)),
        out_shape=jax.ShapeDtypeStruct((B, 1, N), jnp.float32),
        compiler_params=pltpu.CompilerParams(
            dimension_semantics=("arbitrary", "arbitrary")),
    )(fxt, fyt)


def _sc_topk(s):
    mesh = plsc.VectorSubcoreMesh(core_axis_name="c", subcore_axis_name="s")
    chunk_len = N // 4

    @functools.partial(
        pl.kernel,
        mesh=mesh,
        out_type=jax.ShapeDtypeStruct((B, 16), jnp.int32),
        scratch_types=[
            pltpu.VMEM((chunk_len,), jnp.float32),
            pltpu.VMEM((16,), jnp.float32),
            pltpu.VMEM((16,), jnp.int32),
            pltpu.VMEM((4, 16), jnp.float32),
            pltpu.VMEM((4, 16), jnp.int32),
            pltpu.VMEM_SHARED((4, 4, 16), jnp.float32),
            pltpu.VMEM_SHARED((4, 4, 16), jnp.int32),
        ],
        compiler_params=pltpu.CompilerParams(
            use_tc_tiling_on_sc=False, needs_layout_passes=False),
    )
    def k(s_hbm, oi_hbm,
          sv, cand_v, cand_i, merge_v, merge_i, shv, shi):
        ci = lax.axis_index("c")
        si = lax.axis_index("s")
        bl = si // 4
        chunk = si % 4
        batch = ci * 4 + bl
        base = chunk * chunk_len
        pltpu.sync_copy(s_hbm.at[batch, pl.ds(base, chunk_len)], sv)
        lane = lax.iota(jnp.int32, 16)

        def top4(load_row, nrows, fori):
            winners_v, winners_i = [], []
            for _ in range(4):
                prev = list(winners_i)

                def body(j, carry, prev=prev):
                    mv, mi = carry
                    v, iv = load_row(j)
                    for w in prev:
                        v = jnp.where(iv == w, NEG, v)
                    gt = v > mv
                    return jnp.where(gt, v, mv), jnp.where(gt, iv, mi)

                mv = jnp.full((16,), NEG, jnp.float32)
                mi = jnp.zeros((16,), jnp.int32)
                if fori:
                    mv, mi = lax.fori_loop(0, nrows, body, (mv, mi))
                else:
                    for j in range(nrows):
                        mv, mi = body(j, (mv, mi))
                best = jnp.max(mv)
                bi = -jnp.max(jnp.where(mv == best, -mi, jnp.int32(IMIN)))
                winners_v.append(best)
                winners_i.append(bi)
            return winners_v, winners_i

        def load_chunk(j):
            v = sv[pl.ds(j * 16, 16)]
            iv = base + j * 16 + lane
            return v, iv

        wv, wi = top4(load_chunk, chunk_len // 16, fori=True)
        cv = jnp.full((16,), NEG, jnp.float32)
        civ = jnp.zeros((16,), jnp.int32)
        for k4 in range(4):
            cv = jnp.where(lane == k4, wv[k4], cv)
            civ = jnp.where(lane == k4, wi[k4], civ)
        cand_v[...] = cv
        cand_i[...] = civ
        pltpu.sync_copy(cand_v, shv.at[bl, chunk])
        pltpu.sync_copy(cand_i, shi.at[bl, chunk])
        plsc.subcore_barrier()

        @pl.when(chunk == 0)
        def _():
            pltpu.sync_copy(shv.at[bl], merge_v)
            pltpu.sync_copy(shi.at[bl], merge_i)

            def load_merge(j):
                return merge_v[j], merge_i[j]

            _, fin = top4(load_merge, 4, fori=False)
            fiv = jnp.zeros((16,), jnp.int32)
            for k4 in range(KK):
                fiv = jnp.where(lane == k4, fin[k4], fiv)
            cand_i[...] = fiv
            pltpu.sync_copy(cand_i, oi_hbm.at[batch])

    return k(s)


def _gather_body(idx_ref, fx_ref, fy_ref, ox_ref, oy_ref):
    bb = pl.program_id(0)
    kk = pl.program_id(1)
    col = lax.rem(idx_ref[bb, kk], 128)
    lanei = lax.broadcasted_iota(jnp.int32, (1, 1, 128), 2)
    sel = lanei == col
    ox_ref[0, 0] = jnp.sum(jnp.where(sel, fx_ref[0], 0.0), axis=2)
    oy_ref[0, 0] = jnp.sum(jnp.where(sel, fy_ref[0], 0.0), axis=2)


def _gather(idx, fxt, fyt):
    grid_spec = pltpu.PrefetchScalarGridSpec(
        num_scalar_prefetch=1,
        grid=(B, KK),
        in_specs=[
            pl.BlockSpec((1, D, C, 128),
                         lambda b, k, idx: (b, 0, 0, idx[b, k] // 128)),
            pl.BlockSpec((1, D, C, 128),
                         lambda b, k, idx: (b, 0, 0, idx[b, k] // 128)),
        ],
        out_specs=[
            pl.BlockSpec((1, 1, D, C), lambda b, k, idx: (b, k, 0, 0)),
            pl.BlockSpec((1, 1, D, C), lambda b, k, idx: (b, k, 0, 0)),
        ],
    )
    return pl.pallas_call(
        _gather_body,
        grid_spec=grid_spec,
        out_shape=[jax.ShapeDtypeStruct((B, KK, D, C), jnp.float32),
                   jax.ShapeDtypeStruct((B, KK, D, C), jnp.float32)],
    )(idx, fxt, fyt)


def kernel(fx, fy, topk):
    b, c, d, n = fx.shape
    assert (b, c, d, n) == (B, C, D, N), (b, c, d, n)
    fxt = fx.transpose(0, 2, 1, 3)
    fyt = fy.transpose(0, 2, 1, 3)
    s = _scores(fxt, fyt).reshape(b, n)
    idx = _sc_topk(s)[:, :KK]
    gx, gy = _gather(idx, fxt, fyt)
    return gx.transpose(0, 3, 2, 1), gy.transpose(0, 3, 2, 1)


# speedup vs baseline: 5.5015x; 1.0364x over previous
import functools

import jax
import jax.numpy as jnp
from jax import lax
from jax.experimental import pallas as pl
from jax.experimental.pallas import tpu as pltpu
from jax.experimental.pallas import tpu_sc as plsc

NBLK = 1024
B = 8
C = 512
D = 3
N = 4096
KK = B // 2
NEG = float(-3.0e38)
IMIN = -2147483647


def _score_body(fx_ref, fy_ref, s_ref):
    for sub in range(NBLK // 128):
        lo = sub * 128
        fxb = fx_ref[0, :, :, lo:lo + 128]
        fyb = fy_ref[0, :, :, lo:lo + 128]
        acc_m = jnp.zeros((3, 3, 8, 128), jnp.float32)
        acc_x = jnp.zeros((3, 8, 128), jnp.float32)
        acc_y = jnp.zeros((3, 8, 128), jnp.float32)
        for a in range(C // 8):
            xa = fxb[:, 8 * a:8 * a + 8, :]
            ya = fyb[:, 8 * a:8 * a + 8, :]
            acc_m = acc_m + xa[:, None] * ya[None, :]
            acc_x = acc_x + xa
            acc_y = acc_y + ya
        m = jnp.sum(acc_m, axis=2)
        sx = jnp.sum(acc_x, axis=1)
        sy = jnp.sum(acc_y, axis=1)
        inv_c = jnp.float32(1.0 / C)
        mx = sx * inv_c
        my = sy * inv_c
        nx = jnp.sqrt(jnp.sum(mx * mx, axis=0)) + jnp.float32(1e-6)
        ny = jnp.sqrt(jnp.sum(my * my, axis=0)) + jnp.float32(1e-6)
        ux = mx / nx
        uy = my / ny
        s = jnp.sum(ux[:, None, :] * uy[None, :, :] * m, axis=(0, 1))
        s_ref[0, 0, lo:lo + 128] = s


def _scores(fxt, fyt):
    return pl.pallas_call(
        _score_body,
        grid=(B, N // NBLK),
        in_specs=[
            pl.BlockSpec((1, D, C, NBLK), lambda b, nb: (b, 0, 0, nb)),
            pl.BlockSpec((1, D, C, NBLK), lambda b, nb: (b, 0, 0, nb)),
        ],
        out_specs=pl.BlockSpec((1, 1, NBLK), lambda b, nb: (b, 0, nb)),
        out_shape=jax.ShapeDtypeStruct((B, 1, N), jnp.float32),
        compiler_params=pltpu.CompilerParams(
            dimension_semantics=("arbitrary", "arbitrary")),
    )(fxt, fyt)


def _sc_topk(s):
    mesh = plsc.VectorSubcoreMesh(core_axis_name="c", subcore_axis_name="s")
    chunk_len = N // 4

    @functools.partial(
        pl.kernel,
        mesh=mesh,
        out_type=jax.ShapeDtypeStruct((B, 16), jnp.int32),
        scratch_types=[
            pltpu.VMEM((chunk_len,), jnp.float32),
            pltpu.VMEM((16,), jnp.float32),
            pltpu.VMEM((16,), jnp.int32),
            pltpu.VMEM((4, 16), jnp.float32),
            pltpu.VMEM((4, 16), jnp.int32),
            pltpu.VMEM_SHARED((4, 4, 16), jnp.float32),
            pltpu.VMEM_SHARED((4, 4, 16), jnp.int32),
        ],
        compiler_params=pltpu.CompilerParams(
            use_tc_tiling_on_sc=False, needs_layout_passes=False),
    )
    def k(s_hbm, oi_hbm,
          sv, cand_v, cand_i, merge_v, merge_i, shv, shi):
        ci = lax.axis_index("c")
        si = lax.axis_index("s")
        bl = si // 4
        chunk = si % 4
        batch = ci * 4 + bl
        base = chunk * chunk_len
        pltpu.sync_copy(s_hbm.at[batch, pl.ds(base, chunk_len)], sv)
        lane = lax.iota(jnp.int32, 16)

        def top4(load_row, nrows, fori):
            winners_v, winners_i = [], []
            for _ in range(4):
                prev = list(winners_i)

                def body(j, carry, prev=prev):
                    mv, mi = carry
                    v, iv = load_row(j)
                    for w in prev:
                        v = jnp.where(iv == w, NEG, v)
                    gt = v > mv
                    return jnp.where(gt, v, mv), jnp.where(gt, iv, mi)

                mv = jnp.full((16,), NEG, jnp.float32)
                mi = jnp.zeros((16,), jnp.int32)
                if fori:
                    mv, mi = lax.fori_loop(0, nrows, body, (mv, mi))
                else:
                    for j in range(nrows):
                        mv, mi = body(j, (mv, mi))
                best = jnp.max(mv)
                bi = -jnp.max(jnp.where(mv == best, -mi, jnp.int32(IMIN)))
                winners_v.append(best)
                winners_i.append(bi)
            return winners_v, winners_i

        def load_chunk(j):
            v = sv[pl.ds(j * 16, 16)]
            iv = base + j * 16 + lane
            return v, iv

        wv, wi = top4(load_chunk, chunk_len // 16, fori=True)
        cv = jnp.full((16,), NEG, jnp.float32)
        civ = jnp.zeros((16,), jnp.int32)
        for k4 in range(4):
            cv = jnp.where(lane == k4, wv[k4], cv)
            civ = jnp.where(lane == k4, wi[k4], civ)
        cand_v[...] = cv
        cand_i[...] = civ
        pltpu.sync_copy(cand_v, shv.at[bl, chunk])
        pltpu.sync_copy(cand_i, shi.at[bl, chunk])
        plsc.subcore_barrier()

        @pl.when(chunk == 0)
        def _():
            pltpu.sync_copy(shv.at[bl], merge_v)
            pltpu.sync_copy(shi.at[bl], merge_i)

            def load_merge(j):
                return merge_v[j], merge_i[j]

            _, fin = top4(load_merge, 4, fori=False)
            fiv = jnp.zeros((16,), jnp.int32)
            for k4 in range(KK):
                fiv = jnp.where(lane == k4, fin[k4], fiv)
            cand_i[...] = fiv
            pltpu.sync_copy(cand_i, oi_hbm.at[batch])

    return k(s)


def _gather_body(idx_ref, *refs):
    in_refs = refs[:2 * KK]
    ox_ref, oy_ref = refs[2 * KK:]
    bb = pl.program_id(0)
    lanei = lax.broadcasted_iota(jnp.int32, (1, 1, 128), 2)
    for k4 in range(KK):
        col = lax.rem(idx_ref[bb, k4], 128)
        sel = lanei == col
        ox_ref[0, k4] = jnp.sum(jnp.where(sel, in_refs[k4][0], 0.0), axis=2)
        oy_ref[0, k4] = jnp.sum(jnp.where(sel, in_refs[KK + k4][0], 0.0),
                                axis=2)


def _gather(idx, fxt, fyt):
    def slab_spec(k4):
        return pl.BlockSpec((1, D, C, 128),
                            lambda b, idx, k4=k4: (b, 0, 0, idx[b, k4] // 128))

    grid_spec = pltpu.PrefetchScalarGridSpec(
        num_scalar_prefetch=1,
        grid=(B,),
        in_specs=[slab_spec(k4) for k4 in range(KK)] * 2,
        out_specs=[
            pl.BlockSpec((1, KK, D, C), lambda b, idx: (b, 0, 0, 0)),
            pl.BlockSpec((1, KK, D, C), lambda b, idx: (b, 0, 0, 0)),
        ],
    )
    return pl.pallas_call(
        _gather_body,
        grid_spec=grid_spec,
        out_shape=[jax.ShapeDtypeStruct((B, KK, D, C), jnp.float32),
                   jax.ShapeDtypeStruct((B, KK, D, C), jnp.float32)],
    )(idx, *([fxt] * KK), *([fyt] * KK))


def kernel(fx, fy, topk):
    b, c, d, n = fx.shape
    assert (b, c, d, n) == (B, C, D, N), (b, c, d, n)
    fxt = fx.transpose(0, 2, 1, 3)
    fyt = fy.transpose(0, 2, 1, 3)
    s = _scores(fxt, fyt).reshape(b, n)
    idx = _sc_topk(s)[:, :KK]
    gx, gy = _gather(idx, fxt, fyt)
    return gx.transpose(0, 3, 2, 1), gy.transpose(0, 3, 2, 1)
